# Initial kernel scaffold; baseline (speedup 1.0000x reference)
import jax
import jax.numpy as jnp
from jax.experimental import pallas as pl


def kernel(x, edge_index, W1, b1, gamma, beta, prelu_a, W2, b2, Wg, bg, Wf, bf):
    raise NotImplementedError("write your pallas kernel here")


# speedup vs baseline: 167.1182x; 167.1182x over previous
import functools

import jax
import jax.numpy as jnp
from jax import lax
from jax.experimental import pallas as pl
from jax.experimental.pallas import tpu as pltpu
from jax.experimental.pallas import tpu_sc as plsc

NC = 2
NS = 16
VL = 16


def _fill(ref, n, value):
    def body(i, _):
        ref[pl.ds(i * VL, VL)] = jnp.full((VL,), value, jnp.float32)
        return 0
    lax.fori_loop(0, n // VL, body, 0)


def _hist_body(np_, per_tile, chunk, dst_hbm, out_hbm, cnt_sh, dst_v, ones_v,
               zer_v):
    c = lax.axis_index("c")
    s = lax.axis_index("s")
    wid = c * NS + s
    slc = np_ // NS
    _fill(zer_v, slc, 0.0)
    _fill(ones_v, chunk, 1.0)
    pltpu.sync_copy(zer_v, cnt_sh.at[pl.ds(s * slc, slc)])
    plsc.subcore_barrier()
    base = wid * per_tile

    def chunk_body(k, _):
        pltpu.sync_copy(dst_hbm.at[pl.ds(base + k * chunk, chunk)], dst_v)
        pltpu.sync_copy(ones_v, cnt_sh.at[dst_v], add=True)
        return 0

    lax.fori_loop(0, per_tile // chunk, chunk_body, 0)
    plsc.subcore_barrier()
    pltpu.sync_copy(cnt_sh.at[pl.ds(s * slc, slc)], out_hbm.at[c, s])


def _gs_body(np_, per_tile, chunk, src_hbm, dst_hbm, t_hbm, out_hbm, acc_sh,
             t_v, src_v, dst_v, val_v, zer_v):
    c = lax.axis_index("c")
    s = lax.axis_index("s")
    wid = c * NS + s
    slc = np_ // NS
    _fill(zer_v, slc, 0.0)
    pltpu.sync_copy(zer_v, acc_sh.at[pl.ds(s * slc, slc)])
    pltpu.sync_copy(t_hbm, t_v)
    plsc.subcore_barrier()
    base = wid * per_tile

    def chunk_body(k, _):
        b = base + k * chunk
        pltpu.sync_copy(src_hbm.at[pl.ds(b, chunk)], src_v)
        pltpu.sync_copy(dst_hbm.at[pl.ds(b, chunk)], dst_v)
        for j in range(chunk // VL):
            idx = src_v[pl.ds(j * VL, VL)]
            val_v[pl.ds(j * VL, VL)] = plsc.load_gather(t_v, [idx])
        pltpu.sync_copy(val_v, acc_sh.at[dst_v], add=True)
        return 0

    lax.fori_loop(0, per_tile // chunk, chunk_body, 0)
    plsc.subcore_barrier()
    pltpu.sync_copy(acc_sh.at[pl.ds(s * slc, slc)], out_hbm.at[c, s])


def _moments_body(x0_ref, x1_ref, out_ref):
    x0 = x0_ref[...]
    x1 = x1_ref[...]
    out_ref[0:1, :] = jnp.sum(x0, axis=0, keepdims=True)
    out_ref[1:2, :] = jnp.sum(x1, axis=0, keepdims=True)
    out_ref[2:3, :] = jnp.sum(x0 * x0, axis=0, keepdims=True)
    out_ref[3:4, :] = jnp.sum(x1 * x1, axis=0, keepdims=True)
    out_ref[4:5, :] = jnp.sum(x0 * x1, axis=0, keepdims=True)
    out_ref[5:8, :] = jnp.zeros((3, 128), jnp.float32)


def _node_body(p_ref, x0_ref, x1_ref, c0_ref, c1_ref, t_ref, dinv_ref):
    a = p_ref[4, 0]
    x0 = x0_ref[...]
    x1 = x1_ref[...]
    acc = jnp.full(x0.shape, p_ref[4, 1], jnp.float32)
    for j in range(32):
        pre = p_ref[0, j] * x0 + p_ref[1, j] * x1 + p_ref[2, j]
        enc = jnp.where(pre >= 0, pre, a * pre)
        acc = acc + p_ref[3, j] * enc
    deg = c0_ref[...] + c1_ref[...] + 1.0
    dinv = lax.rsqrt(deg)
    t_ref[...] = dinv * acc
    dinv_ref[...] = dinv


def _combine_body(p_ref, a0_ref, a1_ref, t_ref, dinv_ref, out_ref):
    cst = p_ref[0, 0]
    out_ref[...] = dinv_ref[...] * (a0_ref[...] + a1_ref[...] + t_ref[...]) + cst


def kernel(x, edge_index, W1, b1, gamma, beta, prelu_a, W2, b2, Wg, bg, Wf, bf):
    N = x.shape[0]
    E = edge_index.shape[1]
    np_ = ((N + 127) // 128) * 128
    rows = np_ // 128
    slc = np_ // NS
    per_tile = E // (NC * NS)
    chunk = 2000
    f32 = jnp.float32

    pad = np_ - N
    x0p = jnp.pad(x[:, 0], (0, pad)).reshape(rows, 128)
    x1p = jnp.pad(x[:, 1], (0, pad)).reshape(rows, 128)
    src = edge_index[0]
    dst = edge_index[1]

    mesh = plsc.VectorSubcoreMesh(core_axis_name="c", subcore_axis_name="s")
    sc_params = pltpu.CompilerParams(use_tc_tiling_on_sc=False,
                                     needs_layout_passes=False)
    hist = pl.kernel(
        functools.partial(_hist_body, np_, per_tile, chunk),
        out_type=jax.ShapeDtypeStruct((NC, NS, slc), f32),
        mesh=mesh,
        compiler_params=sc_params,
        scratch_types=[
            pltpu.VMEM_SHARED((np_,), f32),
            pltpu.VMEM((chunk,), jnp.int32),
            pltpu.VMEM((chunk,), f32),
            pltpu.VMEM((slc,), f32),
        ],
    )
    cnt = hist(dst)
    cnt_r = cnt.reshape(NC, rows, 128)

    mom = pl.pallas_call(
        _moments_body,
        out_shape=jax.ShapeDtypeStruct((8, 128), f32),
    )(x0p, x1p)
    sums = jnp.sum(mom, axis=1)
    n_f = jnp.float32(N)
    m0, m1 = sums[0] / n_f, sums[1] / n_f
    e00, e11, e01 = sums[2] / n_f, sums[3] / n_f, sums[4] / n_f
    v00 = e00 - m0 * m0
    v01 = e01 - m0 * m1
    v11 = e11 - m1 * m1

    mu = m0 * W1[0] + m1 * W1[1] + b1
    var = v00 * W1[0] ** 2 + 2.0 * v01 * W1[0] * W1[1] + v11 * W1[1] ** 2
    g = gamma * lax.rsqrt(var + 1e-5)
    av = g * W1[0]
    bv = g * W1[1]
    cv = g * (b1 - mu) + beta
    w_eff = (W2 @ Wg @ Wf)[:, 0]
    y_const = (b2 @ Wg @ Wf)[0]
    cst = (bg @ Wf)[0] + bf[0]
    params = (jnp.zeros((5, 32), f32)
              .at[0].set(av).at[1].set(bv).at[2].set(cv).at[3].set(w_eff)
              .at[4, 0].set(prelu_a[0]).at[4, 1].set(y_const))

    t_r, dinv_r = pl.pallas_call(
        _node_body,
        out_shape=[jax.ShapeDtypeStruct((rows, 128), f32),
                   jax.ShapeDtypeStruct((rows, 128), f32)],
        in_specs=[pl.BlockSpec(memory_space=pltpu.SMEM),
                  pl.BlockSpec(memory_space=pltpu.VMEM),
                  pl.BlockSpec(memory_space=pltpu.VMEM),
                  pl.BlockSpec(memory_space=pltpu.VMEM),
                  pl.BlockSpec(memory_space=pltpu.VMEM)],
    )(params, x0p, x1p, cnt_r[0], cnt_r[1])

    gs = pl.kernel(
        functools.partial(_gs_body, np_, per_tile, chunk),
        out_type=jax.ShapeDtypeStruct((NC, NS, slc), f32),
        mesh=mesh,
        compiler_params=sc_params,
        scratch_types=[
            pltpu.VMEM_SHARED((np_,), f32),
            pltpu.VMEM((np_,), f32),
            pltpu.VMEM((chunk,), jnp.int32),
            pltpu.VMEM((chunk,), jnp.int32),
            pltpu.VMEM((chunk,), f32),
            pltpu.VMEM((slc,), f32),
        ],
    )
    acc = gs(src, dst, t_r.reshape(np_))
    acc_r = acc.reshape(NC, rows, 128)

    dparams = jnp.zeros((1, 8), f32).at[0, 0].set(cst)
    scores_r = pl.pallas_call(
        _combine_body,
        out_shape=jax.ShapeDtypeStruct((rows, 128), f32),
        in_specs=[pl.BlockSpec(memory_space=pltpu.SMEM),
                  pl.BlockSpec(memory_space=pltpu.VMEM),
                  pl.BlockSpec(memory_space=pltpu.VMEM),
                  pl.BlockSpec(memory_space=pltpu.VMEM),
                  pl.BlockSpec(memory_space=pltpu.VMEM)],
    )(dparams, acc_r[0], acc_r[1], t_r, dinv_r)
    return scores_r.reshape(np_)[:N]
